# baseline (speedup 1.0000x reference)
import functools

import jax
import jax.numpy as jnp
from jax import lax
from jax.experimental import pallas as pl

DIM = 4096
NUM_EXPERTS = 64
TOP_K = 8
TOKENS_PER_BLOCK = 1536

_NEG_INF = float("-inf")


def _router_block(x_ref, w_ref, logits_ref, weights_ref, indices_ref):
    logits_t = lax.dot_general(
        w_ref[:], x_ref[:], (((1,), (1,)), ((), ())),
        preferred_element_type=jnp.float32,
    )
    logits_ref[:] = logits_t.T

    t = logits_t.shape[1]
    iota = lax.broadcasted_iota(jnp.int32, (NUM_EXPERTS, t), 0)
    cur = logits_t
    vals = []
    idxs = []
    for _ in range(TOP_K):
        m = jnp.max(cur, axis=0, keepdims=True)
        idx = jnp.min(jnp.where(cur == m, iota, NUM_EXPERTS), axis=0, keepdims=True)
        vals.append(m)
        idxs.append(idx)
        cur = jnp.where(iota == idx, _NEG_INF, cur)
    vals = jnp.concatenate(vals, axis=0)
    idxs = jnp.concatenate(idxs, axis=0)

    e = jnp.exp(vals - vals[:1])
    weights_ref[:] = (e / jnp.sum(e, axis=0, keepdims=True)).T
    indices_ref[:] = idxs.T


@jax.jit
def kernel(x, W):
    b, s, d = x.shape
    n_tokens = b * s
    xt = x.reshape(n_tokens, d)

    n_blocks = pl.cdiv(n_tokens, TOKENS_PER_BLOCK)
    logits, weights, indices = pl.pallas_call(
        _router_block,
        grid=(n_blocks,),
        in_specs=[
            pl.BlockSpec((TOKENS_PER_BLOCK, d), lambda i: (i, 0)),
            pl.BlockSpec((NUM_EXPERTS, d), lambda i: (0, 0)),
        ],
        out_specs=[
            pl.BlockSpec((TOKENS_PER_BLOCK, NUM_EXPERTS), lambda i: (i, 0)),
            pl.BlockSpec((TOKENS_PER_BLOCK, TOP_K), lambda i: (i, 0)),
            pl.BlockSpec((TOKENS_PER_BLOCK, TOP_K), lambda i: (i, 0)),
        ],
        out_shape=[
            jax.ShapeDtypeStruct((n_tokens, NUM_EXPERTS), jnp.float32),
            jax.ShapeDtypeStruct((n_tokens, TOP_K), jnp.float32),
            jax.ShapeDtypeStruct((n_tokens, TOP_K), jnp.int32),
        ],
    )(xt, W)

    return (
        weights.reshape(b, s, TOP_K),
        indices.reshape(b, s, TOP_K),
        logits.reshape(b, s, NUM_EXPERTS),
    )


# speedup vs baseline: 1.0155x; 1.0155x over previous
import functools

import jax
import jax.numpy as jnp
from jax import lax
from jax.experimental import pallas as pl

DIM = 4096
NUM_EXPERTS = 64
TOP_K = 8
TOKENS_PER_BLOCK = 1024

_NEG_INF = float("-inf")


def _router_block(x_ref, w_ref, logits_ref, weights_ref, indices_ref):
    logits_t = lax.dot_general(
        w_ref[:], x_ref[:], (((1,), (1,)), ((), ())),
        preferred_element_type=jnp.float32,
    )
    logits_ref[:] = logits_t.T

    t = logits_t.shape[1]
    iota = lax.broadcasted_iota(jnp.int32, (NUM_EXPERTS, t), 0)
    cur = logits_t
    vals = []
    idxs = []
    for _ in range(TOP_K):
        m = jnp.max(cur, axis=0, keepdims=True)
        idx = jnp.min(jnp.where(cur == m, iota, NUM_EXPERTS), axis=0, keepdims=True)
        vals.append(m)
        idxs.append(idx)
        cur = jnp.where(iota == idx, _NEG_INF, cur)
    vals = jnp.concatenate(vals, axis=0)
    idxs = jnp.concatenate(idxs, axis=0)

    e = jnp.exp(vals - vals[:1])
    weights_ref[:] = (e / jnp.sum(e, axis=0, keepdims=True)).T
    indices_ref[:] = idxs.T


@jax.jit
def kernel(x, W):
    b, s, d = x.shape
    n_tokens = b * s
    xt = x.reshape(n_tokens, d)

    n_blocks = pl.cdiv(n_tokens, TOKENS_PER_BLOCK)
    logits, weights, indices = pl.pallas_call(
        _router_block,
        grid=(n_blocks,),
        in_specs=[
            pl.BlockSpec((TOKENS_PER_BLOCK, d), lambda i: (i, 0)),
            pl.BlockSpec((NUM_EXPERTS, d), lambda i: (0, 0)),
        ],
        out_specs=[
            pl.BlockSpec((TOKENS_PER_BLOCK, NUM_EXPERTS), lambda i: (i, 0)),
            pl.BlockSpec((TOKENS_PER_BLOCK, TOP_K), lambda i: (i, 0)),
            pl.BlockSpec((TOKENS_PER_BLOCK, TOP_K), lambda i: (i, 0)),
        ],
        out_shape=[
            jax.ShapeDtypeStruct((n_tokens, NUM_EXPERTS), jnp.float32),
            jax.ShapeDtypeStruct((n_tokens, TOP_K), jnp.float32),
            jax.ShapeDtypeStruct((n_tokens, TOP_K), jnp.int32),
        ],
    )(xt, W)

    return (
        weights.reshape(b, s, TOP_K),
        indices.reshape(b, s, TOP_K),
        logits.reshape(b, s, NUM_EXPERTS),
    )
